# baseline (speedup 1.0000x reference)
import jax
import jax.numpy as jnp
from jax import lax
from jax.experimental import pallas as pl
from jax.experimental.pallas import tpu as pltpu
from jax.experimental.pallas import tpu_sc as plsc

VOCAB = 100000
EMBED_DIM = 128
BATCH = 4096
HIST = 50

NC = 2
NS = 16
NW = NC * NS
TOTAL = BATCH * HIST
B_PER_W = TOTAL // NW
CHUNK = 64
NCHUNK = B_PER_W // CHUNK
NBUF = 10


LEAD = 5


def _body(x_hbm, table_hbm, out_hbm, idx_v, rows_v, *sems):
    gsems = sems[:NBUF]
    wsems = sems[NBUF:]
    sid = lax.axis_index("s")
    wid = sid * NC + lax.axis_index("c")
    base = wid * B_PER_W
    pltpu.sync_copy(x_hbm.at[wid], idx_v)

    def g_start(c, b):
        pltpu.async_copy(table_hbm.at[idx_v.at[c]], rows_v.at[b], gsems[b])

    def g_wait(c, b):
        pltpu.make_async_copy(
            table_hbm.at[idx_v.at[c]], rows_v.at[b], gsems[b]
        ).wait()

    def w_start(c, b):
        pltpu.async_copy(
            rows_v.at[b], out_hbm.at[pl.ds(base + c * CHUNK, CHUNK)], wsems[b]
        )

    def w_wait(c, b):
        pltpu.make_async_copy(
            rows_v.at[b], out_hbm.at[pl.ds(base + c * CHUNK, CHUNK)], wsems[b]
        ).wait()

    for k in range(LEAD):
        g_start(k, k)

    for b in range(NBUF):
        g_wait(b, b)
        w_start(b, b)
        if b >= LEAD:
            w_wait(b - LEAD, (b + LEAD) % NBUF)
        g_start(b + LEAD, (b + LEAD) % NBUF)

    def step(i):
        for b in range(NBUF):
            c = i * NBUF + b
            g_wait(c, b)
            w_start(c, b)
            w_wait(c - LEAD, (b + LEAD) % NBUF)
            g_start(c + LEAD, (b + LEAD) % NBUF)

    pl.loop(1, NCHUNK // NBUF - 1)(step)

    for b in range(NBUF):
        c = NCHUNK - NBUF + b
        g_wait(c, b)
        w_start(c, b)
        w_wait(c - LEAD, (b + LEAD) % NBUF)
        if b < NBUF - LEAD:
            g_start(c + LEAD, (b + LEAD) % NBUF)

    for b in range(NBUF - LEAD, NBUF):
        w_wait(NCHUNK - NBUF + b, b)


@jax.jit
def kernel(x, table):
    x_blocks = x.reshape(NW, NCHUNK, CHUNK)
    mesh = plsc.VectorSubcoreMesh(core_axis_name="c", subcore_axis_name="s")
    out = pl.kernel(
        _body,
        out_type=jax.ShapeDtypeStruct((TOTAL, EMBED_DIM), jnp.float32),
        mesh=mesh,
        scratch_types=[
            pltpu.VMEM((NCHUNK, CHUNK), jnp.int32),
            pltpu.VMEM((NBUF, CHUNK, EMBED_DIM), jnp.float32),
        ] + [pltpu.SemaphoreType.DMA] * (2 * NBUF),
    )(x_blocks, table)
    return out.reshape(BATCH, HIST, EMBED_DIM)


# speedup vs baseline: 1.0002x; 1.0002x over previous
import jax
import jax.numpy as jnp
from jax import lax
from jax.experimental import pallas as pl
from jax.experimental.pallas import tpu as pltpu
from jax.experimental.pallas import tpu_sc as plsc

VOCAB = 100000
EMBED_DIM = 128
BATCH = 4096
HIST = 50

NC = 2
NS = 16
NW = NC * NS
TOTAL = BATCH * HIST
B_PER_W = TOTAL // NW
CHUNK = 64
NCHUNK = B_PER_W // CHUNK
NBUF = 4
WB = 2


def _body(x_hbm, table_hbm, out_hbm, idx_v, rows_v, sh, *sems):
    gsems = sems[:NBUF]
    wsems = sems[NBUF:]
    sid = lax.axis_index("s")
    wid = sid * NC + lax.axis_index("c")
    base = wid * B_PER_W
    pltpu.sync_copy(x_hbm.at[wid], idx_v)

    def g_start(c, b):
        pltpu.async_copy(table_hbm.at[idx_v.at[c]], rows_v.at[b], gsems[b])

    def g_wait(c, b):
        pltpu.make_async_copy(
            table_hbm.at[idx_v.at[c]], rows_v.at[b], gsems[b]
        ).wait()

    def stage(b, s):
        pltpu.sync_copy(rows_v.at[b], sh.at[sid, s])

    def w_start(c, s):
        pltpu.async_copy(
            sh.at[sid, s], out_hbm.at[pl.ds(base + c * CHUNK, CHUNK)], wsems[s]
        )

    def w_wait(c, s):
        pltpu.make_async_copy(
            sh.at[sid, s], out_hbm.at[pl.ds(base + c * CHUNK, CHUNK)], wsems[s]
        ).wait()

    for b in range(NBUF):
        g_start(b, b)

    for b in range(NBUF):
        s = b % WB
        g_wait(b, b)
        if b >= WB:
            w_wait(b - WB, s)
        stage(b, s)
        w_start(b, s)
        g_start(b + NBUF, b)

    def step(i):
        for b in range(NBUF):
            c = i * NBUF + b
            s = b % WB
            g_wait(c, b)
            w_wait(c - WB, s)
            stage(b, s)
            w_start(c, s)
            g_start(c + NBUF, b)

    pl.loop(1, NCHUNK // NBUF - 1)(step)

    for b in range(NBUF):
        c = NCHUNK - NBUF + b
        s = b % WB
        g_wait(c, b)
        w_wait(c - WB, s)
        stage(b, s)
        w_start(c, s)

    for b in range(NBUF - WB, NBUF):
        w_wait(NCHUNK - NBUF + b, b % WB)


@jax.jit
def kernel(x, table):
    x_blocks = x.reshape(NW, NCHUNK, CHUNK)
    mesh = plsc.VectorSubcoreMesh(core_axis_name="c", subcore_axis_name="s")
    out = pl.kernel(
        _body,
        out_type=jax.ShapeDtypeStruct((TOTAL, EMBED_DIM), jnp.float32),
        mesh=mesh,
        scratch_types=[
            pltpu.VMEM((NCHUNK, CHUNK), jnp.int32),
            pltpu.VMEM((NBUF, CHUNK, EMBED_DIM), jnp.float32),
            pltpu.VMEM_SHARED((NS, WB, CHUNK, EMBED_DIM), jnp.float32),
        ] + [pltpu.SemaphoreType.DMA] * (NBUF + WB),
    )(x_blocks, table)
    return out.reshape(BATCH, HIST, EMBED_DIM)


# speedup vs baseline: 1.0045x; 1.0044x over previous
import jax
import jax.numpy as jnp
from jax import lax
from jax.experimental import pallas as pl
from jax.experimental.pallas import tpu as pltpu
from jax.experimental.pallas import tpu_sc as plsc

VOCAB = 100000
EMBED_DIM = 128
BATCH = 4096
HIST = 50

NC = 2
NS = 16
NW = NC * NS
TOTAL = BATCH * HIST
B_PER_W = TOTAL // NW
CHUNK = 128
NCHUNK = B_PER_W // CHUNK
NBUF = 5


def _body(x_hbm, table_hbm, out_hbm, idx_v, rows_v, *sems):
    sid = lax.axis_index("s")
    wid = sid * NC + lax.axis_index("c")
    base = wid * B_PER_W
    pltpu.sync_copy(x_hbm.at[wid], idx_v)

    def gather(c, b):
        pltpu.async_copy(table_hbm.at[idx_v.at[c]], rows_v.at[b], sems[b])

    def wait(c, b):
        pltpu.make_async_copy(
            table_hbm.at[idx_v.at[c]], rows_v.at[b], sems[b]
        ).wait()

    def write(c, b):
        pltpu.sync_copy(rows_v.at[b], out_hbm.at[pl.ds(base + c * CHUNK, CHUNK)])

    for b in range(NBUF):
        gather(b, b)

    def step(i):
        for b in range(NBUF):
            c = i * NBUF + b
            wait(c, b)
            write(c, b)
            gather(c + NBUF, b)

    pl.loop(0, NCHUNK // NBUF - 1)(step)

    for b in range(NBUF):
        c = NCHUNK - NBUF + b
        wait(c, b)
        write(c, b)


@jax.jit
def kernel(x, table):
    x_blocks = x.reshape(NW, NCHUNK, CHUNK)
    mesh = plsc.VectorSubcoreMesh(core_axis_name="c", subcore_axis_name="s")
    out = pl.kernel(
        _body,
        out_type=jax.ShapeDtypeStruct((TOTAL, EMBED_DIM), jnp.float32),
        mesh=mesh,
        scratch_types=[
            pltpu.VMEM((NCHUNK, CHUNK), jnp.int32),
            pltpu.VMEM((NBUF, CHUNK, EMBED_DIM), jnp.float32),
        ] + [pltpu.SemaphoreType.DMA] * NBUF,
    )(x_blocks, table)
    return out.reshape(BATCH, HIST, EMBED_DIM)
